# baseline (speedup 1.0000x reference)
import jax
import jax.numpy as jnp
from jax import lax
from jax.experimental import pallas as pl
from jax.experimental.pallas import tpu as pltpu
from jax.experimental.pallas import tpu_sc as plsc

N = 10000
D = 128
G = 64
H = 256
NC = 2
NS = 16
L = 16
NW = NC * NS
SEG_PER_W = G // NW
NCHUNK = N // L
DCH = D // L
BB = 64
HCH = G // L
CPS = (NCHUNK + NS - 1) // NS


def _sc_pool_body(node_hbm, batch_hbm, out_hbm,
                  bvm, hvm, hall, xb0, xb1, obuf, shsp, sem0, sem1):
    cid = lax.axis_index("c")
    sid = lax.axis_index("s")
    w = sid * NC + cid
    g0 = (w * SEG_PER_W).astype(jnp.int32)

    pltpu.sync_copy(batch_hbm, bvm)

    zeros = jnp.zeros((L,), jnp.int32)

    def cnt_body(i, accs):
        a0, a1, a2 = accs
        c = bvm[pl.ds(i * L, L)]
        a0 = a0 + (c < g0).astype(jnp.int32)
        a1 = a1 + (c < g0 + 1).astype(jnp.int32)
        a2 = a2 + (c < g0 + 2).astype(jnp.int32)
        return (a0, a1, a2)

    a0, a1, a2 = lax.fori_loop(0, NCHUNK, cnt_body, (zeros, zeros, zeros))
    bounds = (jnp.sum(a0), jnp.sum(a1), jnp.sum(a2))

    ninf = jnp.full((L,), -jnp.inf, jnp.float32)

    def blk_start(base, b):
        return pl.multiple_of(jnp.minimum(base + b * BB, N - BB), 8)

    def fire(base, b, xb, sem):
        pltpu.async_copy(node_hbm.at[pl.ds(blk_start(base, b), BB)], xb, sem)

    def drain(xb, sem):
        pltpu.make_async_copy(node_hbm.at[pl.ds(0, BB)], xb, sem).wait()

    def process(xb, s, start, end, acc):
        acc = list(acc)
        for r in range(BB):
            sel = jnp.logical_and(s + r >= start, s + r < end)
            for c in range(DCH):
                v = xb[r, pl.ds(c * L, L)]
                acc[c] = jnp.where(sel, jnp.maximum(acc[c], v), acc[c])
        return tuple(acc)

    for j in range(SEG_PER_W):
        g = g0 + j
        start, end = bounds[j], bounds[j + 1]
        base = pl.multiple_of((start // 8) * 8, 8)
        nb = (end - base + BB - 1) // BB
        nbp = (nb + 1) // 2

        fire(base, 0, xb0, sem0)
        fire(base, 1, xb1, sem1)

        def pair(p, acc, base=base, start=start, end=end):
            b = 2 * p
            drain(xb0, sem0)
            acc = process(xb0, blk_start(base, b), start, end, acc)
            fire(base, b + 2, xb0, sem0)
            drain(xb1, sem1)
            acc = process(xb1, blk_start(base, b + 1), start, end, acc)
            fire(base, b + 3, xb1, sem1)
            return acc

        acc = lax.fori_loop(0, nbp, pair, (ninf,) * DCH)
        drain(xb0, sem0)
        drain(xb1, sem1)
        for c in range(DCH):
            obuf[j, pl.ds(c * L, L)] = acc[c]
        pltpu.sync_copy(obuf.at[j], out_hbm.at[g])


_sc_pool = pl.kernel(
    _sc_pool_body,
    out_type=jax.ShapeDtypeStruct((G, D), jnp.float32),
    mesh=plsc.VectorSubcoreMesh(core_axis_name="c", subcore_axis_name="s",
                                num_cores=NC, num_subcores=NS),
    compiler_params=pltpu.CompilerParams(needs_layout_passes=False),
    scratch_types=[
        pltpu.VMEM((N,), jnp.int32),
        pltpu.VMEM((G,), jnp.int32),
        pltpu.VMEM((NS, G), jnp.int32),
        pltpu.VMEM((BB, D), jnp.float32),
        pltpu.VMEM((BB, D), jnp.float32),
        pltpu.VMEM((SEG_PER_W, D), jnp.float32),
        pltpu.VMEM_SHARED((NS, G), jnp.int32),
        pltpu.SemaphoreType.DMA,
        pltpu.SemaphoreType.DMA,
    ],
)


def _mlp_body(p_ref, w1_ref, b1_ref, w2_ref, b2_ref, out_ref):
    pooled = p_ref[...]
    h = jax.lax.dot_general(pooled, w1_ref[...], (((1,), (1,)), ((), ())),
                            preferred_element_type=jnp.float32)
    h = jnp.maximum(h + b1_ref[...], 0.0)
    y = jax.lax.dot_general(h, w2_ref[...], (((1,), (1,)), ((), ())),
                            preferred_element_type=jnp.float32)
    out_ref[...] = y + b2_ref[...]


def _mlp(pooled, W1, b1r, W2p, b2p):
    return pl.pallas_call(
        _mlp_body,
        out_shape=jax.ShapeDtypeStruct((G, 16), jnp.float32),
    )(pooled, W1, b1r, W2p, b2p)


@jax.jit
def _run(node_emb, batch, W1, b1r, W2p, b2p):
    pooled = _sc_pool(node_emb, batch)
    return _mlp(pooled, W1, b1r, W2p, b2p)


def kernel(node_emb, batch, edge_index, W1, b1, W2, b2):
    T = W2.shape[0]
    W2p = jnp.zeros((16, H), W2.dtype).at[:T].set(W2)
    b2p = jnp.zeros((1, 16), b2.dtype).at[0, :T].set(b2)
    out = _run(node_emb, batch, W1, b1[None, :], W2p, b2p)
    return (out[:, :T], node_emb, edge_index)


# speedup vs baseline: 1.2145x; 1.2145x over previous
import jax
import jax.numpy as jnp
from jax import lax
from jax.experimental import pallas as pl
from jax.experimental.pallas import tpu as pltpu
from jax.experimental.pallas import tpu_sc as plsc

N = 10000
D = 128
G = 64
H = 256
NC = 2
NS = 16
L = 16
NW = NC * NS
SEG_PER_W = G // NW
NCHUNK = N // L
DCH = D // L
BB = 32
HCH = G // L
CPS = (NCHUNK + NS - 1) // NS


def _sc_pool_body(node_hbm, batch_hbm, out_hbm,
                  bvm, hvm, hall, xb0, xb1, obuf, shsp, sem0, sem1):
    cid = lax.axis_index("c")
    sid = lax.axis_index("s")
    w = sid * NC + cid
    g0 = (w * SEG_PER_W).astype(jnp.int32)

    pltpu.sync_copy(batch_hbm, bvm)

    zeros = jnp.zeros((L,), jnp.int32)

    def cnt_body(i, accs):
        a0, a1, a2 = accs
        c = bvm[pl.ds(i * L, L)]
        a0 = a0 + (c < g0).astype(jnp.int32)
        a1 = a1 + (c < g0 + 1).astype(jnp.int32)
        a2 = a2 + (c < g0 + 2).astype(jnp.int32)
        return (a0, a1, a2)

    a0, a1, a2 = lax.fori_loop(0, NCHUNK, cnt_body, (zeros, zeros, zeros))
    bounds = (jnp.sum(a0), jnp.sum(a1), jnp.sum(a2))

    ninf = jnp.full((L,), -jnp.inf, jnp.float32)

    def blk_start(base, b):
        return pl.multiple_of(jnp.minimum(base + b * BB, N - BB), 8)

    def fire(base, b, xb, sem):
        pltpu.async_copy(node_hbm.at[pl.ds(blk_start(base, b), BB)], xb, sem)

    def drain(xb, sem):
        pltpu.make_async_copy(node_hbm.at[pl.ds(0, BB)], xb, sem).wait()

    def process(xb, s, start, end, acc):
        acc = list(acc)
        for r in range(BB):
            sel = jnp.logical_and(s + r >= start, s + r < end)
            for c in range(DCH):
                v = xb[r, pl.ds(c * L, L)]
                acc[c] = jnp.where(sel, jnp.maximum(acc[c], v), acc[c])
        return tuple(acc)

    for j in range(SEG_PER_W):
        g = g0 + j
        start, end = bounds[j], bounds[j + 1]
        base = pl.multiple_of((start // 8) * 8, 8)
        nb = (end - base + BB - 1) // BB
        nbp = (nb + 1) // 2

        fire(base, 0, xb0, sem0)
        fire(base, 1, xb1, sem1)

        def pair(p, acc, base=base, start=start, end=end):
            b = 2 * p
            drain(xb0, sem0)
            acc = process(xb0, blk_start(base, b), start, end, acc)

            @pl.when(p + 1 < nbp)
            def _():
                fire(base, b + 2, xb0, sem0)

            drain(xb1, sem1)
            acc = process(xb1, blk_start(base, b + 1), start, end, acc)

            @pl.when(p + 1 < nbp)
            def _():
                fire(base, b + 3, xb1, sem1)

            return acc

        acc = lax.fori_loop(0, jnp.maximum(nbp, 1), pair, (ninf,) * DCH)
        for c in range(DCH):
            obuf[j, pl.ds(c * L, L)] = acc[c]

    pltpu.sync_copy(obuf, out_hbm.at[pl.ds(g0, SEG_PER_W)])


_sc_pool = pl.kernel(
    _sc_pool_body,
    out_type=jax.ShapeDtypeStruct((G, D), jnp.float32),
    mesh=plsc.VectorSubcoreMesh(core_axis_name="c", subcore_axis_name="s",
                                num_cores=NC, num_subcores=NS),
    compiler_params=pltpu.CompilerParams(needs_layout_passes=False),
    scratch_types=[
        pltpu.VMEM((N,), jnp.int32),
        pltpu.VMEM((G,), jnp.int32),
        pltpu.VMEM((NS, G), jnp.int32),
        pltpu.VMEM((BB, D), jnp.float32),
        pltpu.VMEM((BB, D), jnp.float32),
        pltpu.VMEM((SEG_PER_W, D), jnp.float32),
        pltpu.VMEM_SHARED((NS, G), jnp.int32),
        pltpu.SemaphoreType.DMA,
        pltpu.SemaphoreType.DMA,
    ],
)


def _mlp_body(p_ref, w1_ref, b1_ref, w2_ref, b2_ref, out_ref):
    pooled = p_ref[...]
    h = jax.lax.dot_general(pooled, w1_ref[...], (((1,), (1,)), ((), ())),
                            preferred_element_type=jnp.float32)
    h = jnp.maximum(h + b1_ref[...], 0.0)
    y = jax.lax.dot_general(h, w2_ref[...], (((1,), (1,)), ((), ())),
                            preferred_element_type=jnp.float32)
    out_ref[...] = y + b2_ref[...]


def _mlp(pooled, W1, b1r, W2r, b2r):
    return pl.pallas_call(
        _mlp_body,
        out_shape=jax.ShapeDtypeStruct((G, 10), jnp.float32),
    )(pooled, W1, b1r, W2r, b2r)


@jax.jit
def _run(node_emb, batch, W1, b1r, W2p, b2p):
    pooled = _sc_pool(node_emb, batch)
    return _mlp(pooled, W1, b1r, W2p, b2p)


def kernel(node_emb, batch, edge_index, W1, b1, W2, b2):
    out = _run(node_emb, batch, W1, b1[None, :], W2, b2[None, :])
    return (out, node_emb, edge_index)


# speedup vs baseline: 1.3566x; 1.1170x over previous
import jax
import jax.numpy as jnp
from jax import lax
from jax.experimental import pallas as pl
from jax.experimental.pallas import tpu as pltpu
from jax.experimental.pallas import tpu_sc as plsc

N = 10000
D = 128
G = 64
H = 256
NC = 2
NS = 16
L = 16
NW = NC * NS
SEG_PER_W = G // NW
NCHUNK = N // L
DCH = D // L
BB = 32
HCH = G // L
CPS = (NCHUNK + NS - 1) // NS


def _sc_pool_body(node_hbm, batch_hbm, out_hbm,
                  bvm, hvm, hall, xb0, xb1, obuf, shsp, sem0, sem1):
    cid = lax.axis_index("c")
    sid = lax.axis_index("s")
    w = sid * NC + cid
    g0 = (w * SEG_PER_W).astype(jnp.int32)

    pltpu.sync_copy(batch_hbm, bvm)

    zeros = jnp.zeros((L,), jnp.int32)

    def cnt_body(i, accs):
        a0, a1, a2 = accs
        c = bvm[pl.ds(i * L, L)]
        a0 = a0 + (c < g0).astype(jnp.int32)
        a1 = a1 + (c < g0 + 1).astype(jnp.int32)
        a2 = a2 + (c < g0 + 2).astype(jnp.int32)
        return (a0, a1, a2)

    a0, a1, a2 = lax.fori_loop(0, NCHUNK, cnt_body, (zeros, zeros, zeros))
    bounds = (jnp.sum(a0), jnp.sum(a1), jnp.sum(a2))

    ninf = jnp.full((L,), -jnp.inf, jnp.float32)

    def blk_start(base, b):
        return pl.multiple_of(jnp.minimum(base + b * BB, N - BB), 8)

    def fire(base, b, xb, sem):
        pltpu.async_copy(node_hbm.at[pl.ds(blk_start(base, b), BB)], xb, sem)

    def drain(xb, sem):
        pltpu.make_async_copy(node_hbm.at[pl.ds(0, BB)], xb, sem).wait()

    def process(xb, s, start, end, acc):
        def row_body(r, a):
            sel = jnp.logical_and(s + r >= start, s + r < end)
            a = list(a)
            for c in range(DCH):
                v = xb[r, pl.ds(c * L, L)]
                a[c] = jnp.where(sel, jnp.maximum(a[c], v), a[c])
            return tuple(a)
        return lax.fori_loop(0, BB, row_body, acc)

    for j in range(SEG_PER_W):
        g = g0 + j
        start, end = bounds[j], bounds[j + 1]
        base = pl.multiple_of((start // 8) * 8, 8)
        nb = (end - base + BB - 1) // BB
        nbp = (nb + 1) // 2

        fire(base, 0, xb0, sem0)
        fire(base, 1, xb1, sem1)

        def pair(p, acc, base=base, start=start, end=end):
            b = 2 * p
            drain(xb0, sem0)
            acc = process(xb0, blk_start(base, b), start, end, acc)

            @pl.when(p + 1 < nbp)
            def _():
                fire(base, b + 2, xb0, sem0)

            drain(xb1, sem1)
            acc = process(xb1, blk_start(base, b + 1), start, end, acc)

            @pl.when(p + 1 < nbp)
            def _():
                fire(base, b + 3, xb1, sem1)

            return acc

        acc = lax.fori_loop(0, jnp.maximum(nbp, 1), pair, (ninf,) * DCH)
        for c in range(DCH):
            obuf[j, pl.ds(c * L, L)] = acc[c]

    pltpu.sync_copy(obuf, out_hbm.at[pl.ds(g0, SEG_PER_W)])


_sc_pool = pl.kernel(
    _sc_pool_body,
    out_type=jax.ShapeDtypeStruct((G, D), jnp.float32),
    mesh=plsc.VectorSubcoreMesh(core_axis_name="c", subcore_axis_name="s",
                                num_cores=NC, num_subcores=NS),
    compiler_params=pltpu.CompilerParams(needs_layout_passes=False),
    scratch_types=[
        pltpu.VMEM((N,), jnp.int32),
        pltpu.VMEM((G,), jnp.int32),
        pltpu.VMEM((NS, G), jnp.int32),
        pltpu.VMEM((BB, D), jnp.float32),
        pltpu.VMEM((BB, D), jnp.float32),
        pltpu.VMEM((SEG_PER_W, D), jnp.float32),
        pltpu.VMEM_SHARED((NS, G), jnp.int32),
        pltpu.SemaphoreType.DMA,
        pltpu.SemaphoreType.DMA,
    ],
)


def _mlp_body(p_ref, w1_ref, b1_ref, w2_ref, b2_ref, out_ref):
    pooled = p_ref[...]
    h = jax.lax.dot_general(pooled, w1_ref[...], (((1,), (1,)), ((), ())),
                            preferred_element_type=jnp.float32)
    h = jnp.maximum(h + b1_ref[...], 0.0)
    y = jax.lax.dot_general(h, w2_ref[...], (((1,), (1,)), ((), ())),
                            preferred_element_type=jnp.float32)
    out_ref[...] = y + b2_ref[...]


def _mlp(pooled, W1, b1r, W2r, b2r):
    return pl.pallas_call(
        _mlp_body,
        out_shape=jax.ShapeDtypeStruct((G, 10), jnp.float32),
    )(pooled, W1, b1r, W2r, b2r)


@jax.jit
def _run(node_emb, batch, W1, b1r, W2p, b2p):
    pooled = _sc_pool(node_emb, batch)
    return _mlp(pooled, W1, b1r, W2p, b2p)


def kernel(node_emb, batch, edge_index, W1, b1, W2, b2):
    out = _run(node_emb, batch, W1, b1[None, :], W2, b2[None, :])
    return (out, node_emb, edge_index)
